# baseline (speedup 1.0000x reference)
import functools

import jax
import jax.numpy as jnp
from jax import lax
from jax.experimental import pallas as pl
from jax.experimental.pallas import tpu as pltpu
from jax.experimental.pallas import tpu_sc as plsc

B, K, N, DIM, C_IN = 4, 1024, 2048, 3, 125
IN_SIZE = C_IN + DIM
H1 = H2 = C_OUT = 256
NB = 16

NC, NS = 2, 16
NW = NC * NS
NCHUNK = N // 16
ILV = 4
UNROLL = 2
SPLIT = 4


def _make_fused_body(nbatch):
    nq = nbatch * K
    qpw = nq // NW
    tpb = NW // nbatch

    def body(pts_hbm, keys_hbm, table_hbm, out_hbm, pts_v, keys_v,
             colbuf, rows_v0, rows_v1, sem0, sem1):
        c = lax.axis_index("c")
        s = lax.axis_index("s")
        wid = s * NC + c
        b = wid // tpb
        pltpu.sync_copy(pts_hbm.at[b], pts_v)
        pltpu.sync_copy(keys_hbm.at[wid], keys_v)

        def gbody(gi, _):
            g0 = gi * 16
            kxv = keys_v[pl.ds(g0, 16)]
            kyv = keys_v[pl.ds(qpw + g0, 16)]
            kzv = keys_v[pl.ds(2 * qpw + g0, 16)]
            for j0 in range(0, 16, ILV):
                kq = [(kxv[j0 + t], kyv[j0 + t], kzv[j0 + t])
                      for t in range(ILV)]

                def cbody(ci, carry):
                    off = ci * 16
                    px = pts_v[pl.ds(off, 16)]
                    py = pts_v[pl.ds(N + off, 16)]
                    pz = pts_v[pl.ds(2 * N + off, 16)]
                    idxv = off + lax.iota(jnp.int32, 16)
                    nxt = []
                    for t in range(ILV):
                        kx, ky, kz = kq[t]
                        rd, ri = carry[2 * t], carry[2 * t + 1]
                        dx = px - kx
                        dy = py - ky
                        dz = pz - kz
                        d = dx * dx + dy * dy + dz * dz
                        sk, sv = plsc.sort_key_val(d, idxv, descending=True)
                        take = sk < rd
                        md = jnp.where(take, sk, rd)
                        mi = jnp.where(take, sv, ri)
                        nxt += list(plsc.sort_key_val(md, mi))
                    return tuple(nxt)

                inf16 = jnp.full((16,), jnp.inf, jnp.float32)
                z16 = jnp.zeros((16,), jnp.int32)
                res = lax.fori_loop(0, NCHUNK, cbody, (inf16, z16) * ILV,
                                    unroll=UNROLL)
                lanes = lax.iota(jnp.int32, 16) * qpw
                for t in range(ILV):
                    plsc.store_scatter(colbuf, [lanes + (g0 + j0 + t)],
                                       res[2 * t + 1] + b * N)
            return 0

        lax.fori_loop(0, qpw // 16, gbody, 0)

        def start(j, rows, sem):
            idx = colbuf.at[pl.ds(j * qpw, qpw)]
            pltpu.async_copy(table_hbm.at[idx], rows, sem)

        def drain(rows, sem):
            pltpu.make_async_copy(table_hbm.at[pl.ds(0, qpw)], rows,
                                  sem).wait()

        def out_slice(j):
            return out_hbm.at[pl.ds(j * nq + wid * qpw, qpw)]

        start(0, rows_v0, sem0)

        def gather_loop(i, _):
            ja = 2 * i
            jb = 2 * i + 1
            jc = lax.rem(2 * i + 2, NB)
            start(jb, rows_v1, sem1)
            drain(rows_v0, sem0)
            pltpu.sync_copy(rows_v0, out_slice(ja))
            start(jc, rows_v0, sem0)
            drain(rows_v1, sem1)
            pltpu.sync_copy(rows_v1, out_slice(jb))
            return 0

        lax.fori_loop(0, NB // 2, gather_loop, 0)
        drain(rows_v0, sem0)
        pltpu.sync_copy(rows_v0, out_slice(0))

    return body, nq, qpw


def _fused_call(pts_t, keys_g, table, nbatch):
    body, nq, qpw = _make_fused_body(nbatch)
    mesh = plsc.VectorSubcoreMesh(core_axis_name="c", subcore_axis_name="s",
                                  num_cores=NC, num_subcores=NS)
    fn = functools.partial(
        pl.kernel,
        out_type=jax.ShapeDtypeStruct((nq * NB, IN_SIZE), jnp.float32),
        mesh=mesh,
        compiler_params=pltpu.CompilerParams(needs_layout_passes=False),
        scratch_types=[
            pltpu.VMEM((DIM * N,), jnp.float32),
            pltpu.VMEM((DIM * qpw,), jnp.float32),
            pltpu.VMEM((NB * qpw,), jnp.int32),
            pltpu.VMEM((qpw, IN_SIZE), jnp.float32),
            pltpu.VMEM((qpw, IN_SIZE), jnp.float32),
            pltpu.SemaphoreType.DMA,
            pltpu.SemaphoreType.DMA,
        ],
    )(body)
    return fn(pts_t, keys_g, table)


QB = 256


def _mlp_body(g_ref, kn_ref, w1_ref, w1k_ref, b1_ref, w2_ref, b2_ref,
              w3_ref, b3_ref, out_ref):
    f32 = jnp.float32
    kt = jnp.dot(kn_ref[...], w1k_ref[...], preferred_element_type=f32)
    kt = kt + b1_ref[...]
    acc = jnp.full((QB, C_OUT), -jnp.inf, f32)
    for j in range(NB):
        x = g_ref[j]
        h = jnp.dot(x, w1_ref[...], preferred_element_type=f32) + kt
        h = jnp.maximum(h, 0.0)
        h = jnp.dot(h, w2_ref[...], preferred_element_type=f32) + b2_ref[...]
        h = jnp.maximum(h, 0.0)
        o = jnp.dot(h, w3_ref[...], preferred_element_type=f32)
        acc = jnp.maximum(acc, o)
    out_ref[...] = acc + b3_ref[...]


def _mlp_call(g, kn, W1, W1k, b1, W2, b2, W3, b3):
    nq = g.shape[1]
    grid = (nq // QB,)
    return pl.pallas_call(
        _mlp_body,
        grid=grid,
        in_specs=[
            pl.BlockSpec((NB, QB, IN_SIZE), lambda i: (0, i, 0)),
            pl.BlockSpec((QB, 8), lambda i: (i, 0)),
            pl.BlockSpec((IN_SIZE, H1), lambda i: (0, 0)),
            pl.BlockSpec((8, H1), lambda i: (0, 0)),
            pl.BlockSpec((1, H1), lambda i: (0, 0)),
            pl.BlockSpec((H1, H2), lambda i: (0, 0)),
            pl.BlockSpec((1, H2), lambda i: (0, 0)),
            pl.BlockSpec((H2, C_OUT), lambda i: (0, 0)),
            pl.BlockSpec((1, C_OUT), lambda i: (0, 0)),
        ],
        out_specs=pl.BlockSpec((QB, C_OUT), lambda i: (i, 0)),
        out_shape=jax.ShapeDtypeStruct((nq, C_OUT), jnp.float32),
    )(g, kn, W1, W1k, b1, W2, b2, W3, b3)


def kernel(keys, points, feats, W1, b1, W2, b2, W3, b3):
    f32 = jnp.float32
    nb = B // SPLIT
    tpb = NW // nb
    qpw = nb * K // NW

    W1k = jnp.concatenate([W1[:DIM], jnp.zeros((8 - DIM, H1), f32)], axis=0)
    b1r, b2r, b3r = b1.reshape(1, H1), b2.reshape(1, H2), b3.reshape(1, C_OUT)

    outs = []
    for h in range(SPLIT):
        kh = lax.slice_in_dim(keys, h * nb, (h + 1) * nb, axis=0)
        ph = lax.slice_in_dim(points, h * nb, (h + 1) * nb, axis=0)
        fh = lax.slice_in_dim(feats, h * nb, (h + 1) * nb, axis=0)
        nq = nb * K
        pts_t = ph.transpose(0, 2, 1).reshape(nb, DIM * N)
        keys_g = (kh.reshape(nb, tpb, qpw, DIM)
                  .transpose(0, 1, 3, 2)
                  .reshape(NW, DIM * qpw))
        table = jnp.concatenate([ph, fh], axis=2).reshape(nb * N, IN_SIZE)
        g = _fused_call(pts_t, keys_g, table, nb).reshape(NB, nq, IN_SIZE)
        kflat = kh.reshape(nq, DIM)
        kn = jnp.concatenate([-kflat, jnp.zeros((nq, 8 - DIM), f32)], axis=1)
        outs.append(_mlp_call(g, kn, W1, W1k, b1r, W2, b2r, W3, b3r))

    out = jnp.concatenate(outs, axis=0) if SPLIT > 1 else outs[0]
    return out.reshape(B, K, C_OUT)


# speedup vs baseline: 1.1393x; 1.1393x over previous
import functools

import jax
import jax.numpy as jnp
from jax import lax
from jax.experimental import pallas as pl
from jax.experimental.pallas import tpu as pltpu
from jax.experimental.pallas import tpu_sc as plsc

B, K, N, DIM, C_IN = 4, 1024, 2048, 3, 125
IN_SIZE = C_IN + DIM
H1 = H2 = C_OUT = 256
NB = 16

NC, NS = 2, 16
NW = NC * NS
NCHUNK = N // 16
ILV = 4
UNROLL = 2
SPLIT = 2


def _make_fused_body(nbatch):
    nq = nbatch * K
    qpw = nq // NW
    tpb = NW // nbatch

    def body(pts_hbm, keys_hbm, table_hbm, out_hbm, pts_v, keys_v,
             colbuf, rows_v0, rows_v1, sem0, sem1):
        c = lax.axis_index("c")
        s = lax.axis_index("s")
        wid = s * NC + c
        b = wid // tpb
        pltpu.sync_copy(pts_hbm.at[b], pts_v)
        pltpu.sync_copy(keys_hbm.at[wid], keys_v)

        def gbody(gi, _):
            g0 = gi * 16
            kxv = keys_v[pl.ds(g0, 16)]
            kyv = keys_v[pl.ds(qpw + g0, 16)]
            kzv = keys_v[pl.ds(2 * qpw + g0, 16)]
            for j0 in range(0, 16, ILV):
                kq = [(kxv[j0 + t], kyv[j0 + t], kzv[j0 + t])
                      for t in range(ILV)]

                HOFF = (NCHUNK // 2) * 16

                def cbody(ci, carry):
                    off = ci * 16
                    pts = []
                    for hof in (0, HOFF):
                        pts.append((pts_v[pl.ds(hof + off, 16)],
                                    pts_v[pl.ds(N + hof + off, 16)],
                                    pts_v[pl.ds(2 * N + hof + off, 16)],
                                    hof + off + lax.iota(jnp.int32, 16)))
                    nxt = []
                    for t in range(ILV):
                        kx, ky, kz = kq[t]
                        for hi in range(2):
                            px, py, pz, idxv = pts[hi]
                            rd = carry[4 * t + 2 * hi]
                            ri = carry[4 * t + 2 * hi + 1]
                            dx = px - kx
                            dy = py - ky
                            dz = pz - kz
                            d = dx * dx + dy * dy + dz * dz
                            sk, sv = plsc.sort_key_val(d, idxv,
                                                       descending=True)
                            take = sk < rd
                            md = jnp.where(take, sk, rd)
                            mi = jnp.where(take, sv, ri)
                            nxt += list(plsc.sort_key_val(md, mi))
                    return tuple(nxt)

                inf16 = jnp.full((16,), jnp.inf, jnp.float32)
                z16 = jnp.zeros((16,), jnp.int32)
                res = lax.fori_loop(0, NCHUNK // 2, cbody,
                                    (inf16, z16) * (2 * ILV),
                                    unroll=UNROLL)
                lanes = lax.iota(jnp.int32, 16) * qpw
                for t in range(ILV):
                    rda, ria = res[4 * t], res[4 * t + 1]
                    rdb = lax.rev(res[4 * t + 2], (0,))
                    rib = lax.rev(res[4 * t + 3], (0,))
                    take = rdb < rda
                    ri = jnp.where(take, rib, ria)
                    plsc.store_scatter(colbuf, [lanes + (g0 + j0 + t)],
                                       ri + b * N)
            return 0

        lax.fori_loop(0, qpw // 16, gbody, 0)

        def start(j, rows, sem):
            idx = colbuf.at[pl.ds(j * qpw, qpw)]
            pltpu.async_copy(table_hbm.at[idx], rows, sem)

        def drain(rows, sem):
            pltpu.make_async_copy(table_hbm.at[pl.ds(0, qpw)], rows,
                                  sem).wait()

        def out_slice(j):
            return out_hbm.at[pl.ds(j * nq + wid * qpw, qpw)]

        start(0, rows_v0, sem0)

        def gather_loop(i, _):
            ja = 2 * i
            jb = 2 * i + 1
            jc = lax.rem(2 * i + 2, NB)
            start(jb, rows_v1, sem1)
            drain(rows_v0, sem0)
            pltpu.sync_copy(rows_v0, out_slice(ja))
            start(jc, rows_v0, sem0)
            drain(rows_v1, sem1)
            pltpu.sync_copy(rows_v1, out_slice(jb))
            return 0

        lax.fori_loop(0, NB // 2, gather_loop, 0)
        drain(rows_v0, sem0)
        pltpu.sync_copy(rows_v0, out_slice(0))

    return body, nq, qpw


def _fused_call(pts_t, keys_g, table, nbatch):
    body, nq, qpw = _make_fused_body(nbatch)
    mesh = plsc.VectorSubcoreMesh(core_axis_name="c", subcore_axis_name="s",
                                  num_cores=NC, num_subcores=NS)
    fn = functools.partial(
        pl.kernel,
        out_type=jax.ShapeDtypeStruct((nq * NB, IN_SIZE), jnp.float32),
        mesh=mesh,
        compiler_params=pltpu.CompilerParams(needs_layout_passes=False),
        scratch_types=[
            pltpu.VMEM((DIM * N,), jnp.float32),
            pltpu.VMEM((DIM * qpw,), jnp.float32),
            pltpu.VMEM((NB * qpw,), jnp.int32),
            pltpu.VMEM((qpw, IN_SIZE), jnp.float32),
            pltpu.VMEM((qpw, IN_SIZE), jnp.float32),
            pltpu.SemaphoreType.DMA,
            pltpu.SemaphoreType.DMA,
        ],
    )(body)
    return fn(pts_t, keys_g, table)


QB = 256


def _mlp_body(g_ref, kn_ref, w1_ref, w1k_ref, b1_ref, w2_ref, b2_ref,
              w3_ref, b3_ref, out_ref):
    f32 = jnp.float32
    kt = jnp.dot(kn_ref[...], w1k_ref[...], preferred_element_type=f32)
    kt = kt + b1_ref[...]
    acc = jnp.full((QB, C_OUT), -jnp.inf, f32)
    for j in range(NB):
        x = g_ref[j]
        h = jnp.dot(x, w1_ref[...], preferred_element_type=f32) + kt
        h = jnp.maximum(h, 0.0)
        h = jnp.dot(h, w2_ref[...], preferred_element_type=f32) + b2_ref[...]
        h = jnp.maximum(h, 0.0)
        o = jnp.dot(h, w3_ref[...], preferred_element_type=f32)
        acc = jnp.maximum(acc, o)
    out_ref[...] = acc + b3_ref[...]


def _mlp_call(g, kn, W1, W1k, b1, W2, b2, W3, b3):
    nq = g.shape[1]
    grid = (nq // QB,)
    return pl.pallas_call(
        _mlp_body,
        grid=grid,
        in_specs=[
            pl.BlockSpec((NB, QB, IN_SIZE), lambda i: (0, i, 0)),
            pl.BlockSpec((QB, 8), lambda i: (i, 0)),
            pl.BlockSpec((IN_SIZE, H1), lambda i: (0, 0)),
            pl.BlockSpec((8, H1), lambda i: (0, 0)),
            pl.BlockSpec((1, H1), lambda i: (0, 0)),
            pl.BlockSpec((H1, H2), lambda i: (0, 0)),
            pl.BlockSpec((1, H2), lambda i: (0, 0)),
            pl.BlockSpec((H2, C_OUT), lambda i: (0, 0)),
            pl.BlockSpec((1, C_OUT), lambda i: (0, 0)),
        ],
        out_specs=pl.BlockSpec((QB, C_OUT), lambda i: (i, 0)),
        out_shape=jax.ShapeDtypeStruct((nq, C_OUT), jnp.float32),
    )(g, kn, W1, W1k, b1, W2, b2, W3, b3)


def kernel(keys, points, feats, W1, b1, W2, b2, W3, b3):
    f32 = jnp.float32
    nb = B // SPLIT
    tpb = NW // nb
    qpw = nb * K // NW

    W1k = jnp.concatenate([W1[:DIM], jnp.zeros((8 - DIM, H1), f32)], axis=0)
    b1r, b2r, b3r = b1.reshape(1, H1), b2.reshape(1, H2), b3.reshape(1, C_OUT)

    outs = []
    for h in range(SPLIT):
        kh = lax.slice_in_dim(keys, h * nb, (h + 1) * nb, axis=0)
        ph = lax.slice_in_dim(points, h * nb, (h + 1) * nb, axis=0)
        fh = lax.slice_in_dim(feats, h * nb, (h + 1) * nb, axis=0)
        nq = nb * K
        pts_t = ph.transpose(0, 2, 1).reshape(nb, DIM * N)
        keys_g = (kh.reshape(nb, tpb, qpw, DIM)
                  .transpose(0, 1, 3, 2)
                  .reshape(NW, DIM * qpw))
        table = jnp.concatenate([ph, fh], axis=2).reshape(nb * N, IN_SIZE)
        g = _fused_call(pts_t, keys_g, table, nb).reshape(NB, nq, IN_SIZE)
        kflat = kh.reshape(nq, DIM)
        kn = jnp.concatenate([-kflat, jnp.zeros((nq, 8 - DIM), f32)], axis=1)
        outs.append(_mlp_call(g, kn, W1, W1k, b1r, W2, b2r, W3, b3r))

    out = jnp.concatenate(outs, axis=0) if SPLIT > 1 else outs[0]
    return out.reshape(B, K, C_OUT)


# speedup vs baseline: 1.1655x; 1.0229x over previous
import functools

import jax
import jax.numpy as jnp
from jax import lax
from jax.experimental import pallas as pl
from jax.experimental.pallas import tpu as pltpu
from jax.experimental.pallas import tpu_sc as plsc

B, K, N, DIM, C_IN = 4, 1024, 2048, 3, 125
IN_SIZE = C_IN + DIM
H1 = H2 = C_OUT = 256
NB = 16

NC, NS = 2, 16
NW = NC * NS
NCHUNK = N // 16
ILV = 4
UNROLL = 2
SPLIT = 2


def _make_fused_body(nbatch):
    nq = nbatch * K
    qpw = nq // NW
    tpb = NW // nbatch

    def body(pts_hbm, keys_hbm, table_hbm, out_hbm, pts_v, keys_v,
             colbuf, rows_v0, rows_v1, sem0, sem1):
        c = lax.axis_index("c")
        s = lax.axis_index("s")
        wid = s * NC + c
        b = wid // tpb
        pltpu.sync_copy(pts_hbm.at[b], pts_v)
        pltpu.sync_copy(keys_hbm.at[wid], keys_v)

        def gbody(gi, _):
            g0 = gi * 16
            kxv = keys_v[pl.ds(g0, 16)]
            kyv = keys_v[pl.ds(qpw + g0, 16)]
            kzv = keys_v[pl.ds(2 * qpw + g0, 16)]
            for j0 in range(0, 16, ILV):
                kq = [(kxv[j0 + t], kyv[j0 + t], kzv[j0 + t])
                      for t in range(ILV)]

                def cbody(ci, carry):
                    off = ci * 16
                    px = pts_v[pl.ds(off, 16)]
                    py = pts_v[pl.ds(N + off, 16)]
                    pz = pts_v[pl.ds(2 * N + off, 16)]
                    idxv = off + lax.iota(jnp.int32, 16)
                    nxt = []
                    for t in range(ILV):
                        kx, ky, kz = kq[t]
                        rd, ri = carry[2 * t], carry[2 * t + 1]
                        dx = px - kx
                        dy = py - ky
                        dz = pz - kz
                        d = dx * dx + dy * dy + dz * dz
                        sk, sv = plsc.sort_key_val(d, idxv, descending=True)
                        take = sk < rd
                        md = jnp.where(take, sk, rd)
                        mi = jnp.where(take, sv, ri)
                        nxt += list(plsc.sort_key_val(md, mi))
                    return tuple(nxt)

                inf16 = jnp.full((16,), jnp.inf, jnp.float32)
                z16 = jnp.zeros((16,), jnp.int32)
                res = lax.fori_loop(0, NCHUNK, cbody, (inf16, z16) * ILV,
                                    unroll=UNROLL)
                lanes = lax.iota(jnp.int32, 16) * qpw
                for t in range(ILV):
                    plsc.store_scatter(colbuf, [lanes + (g0 + j0 + t)],
                                       res[2 * t + 1] + b * N)
            return 0

        lax.fori_loop(0, qpw // 16, gbody, 0)

        def start(j, rows, sem):
            idx = colbuf.at[pl.ds(j * qpw, qpw)]
            pltpu.async_copy(table_hbm.at[idx], rows, sem)

        def drain(rows, sem):
            pltpu.make_async_copy(table_hbm.at[pl.ds(0, qpw)], rows,
                                  sem).wait()

        def out_slice(j):
            return out_hbm.at[pl.ds(j * nq + wid * qpw, qpw)]

        start(0, rows_v0, sem0)

        def gather_loop(i, _):
            ja = 2 * i
            jb = 2 * i + 1
            jc = lax.rem(2 * i + 2, NB)
            start(jb, rows_v1, sem1)
            drain(rows_v0, sem0)
            pltpu.sync_copy(rows_v0, out_slice(ja))
            start(jc, rows_v0, sem0)
            drain(rows_v1, sem1)
            pltpu.sync_copy(rows_v1, out_slice(jb))
            return 0

        lax.fori_loop(0, NB // 2, gather_loop, 0)
        drain(rows_v0, sem0)
        pltpu.sync_copy(rows_v0, out_slice(0))

    return body, nq, qpw


def _fused_call(pts_t, keys_g, table, nbatch):
    body, nq, qpw = _make_fused_body(nbatch)
    mesh = plsc.VectorSubcoreMesh(core_axis_name="c", subcore_axis_name="s",
                                  num_cores=NC, num_subcores=NS)
    fn = functools.partial(
        pl.kernel,
        out_type=jax.ShapeDtypeStruct((nq * NB, IN_SIZE), jnp.float32),
        mesh=mesh,
        compiler_params=pltpu.CompilerParams(needs_layout_passes=False),
        scratch_types=[
            pltpu.VMEM((DIM * N,), jnp.float32),
            pltpu.VMEM((DIM * qpw,), jnp.float32),
            pltpu.VMEM((NB * qpw,), jnp.int32),
            pltpu.VMEM((qpw, IN_SIZE), jnp.float32),
            pltpu.VMEM((qpw, IN_SIZE), jnp.float32),
            pltpu.SemaphoreType.DMA,
            pltpu.SemaphoreType.DMA,
        ],
    )(body)
    return fn(pts_t, keys_g, table)


QB = 256


def _mlp_body(g_ref, kn_ref, w1_ref, w1k_ref, b1_ref, w2_ref, b2_ref,
              w3_ref, b3_ref, out_ref):
    f32 = jnp.float32
    kt = jnp.dot(kn_ref[...], w1k_ref[...], preferred_element_type=f32)
    kt = kt + b1_ref[...]
    acc = jnp.full((QB, C_OUT), -jnp.inf, f32)
    for j in range(NB):
        x = g_ref[j]
        h = jnp.dot(x, w1_ref[...], preferred_element_type=f32) + kt
        h = jnp.maximum(h, 0.0)
        h = jnp.dot(h, w2_ref[...], preferred_element_type=f32) + b2_ref[...]
        h = jnp.maximum(h, 0.0)
        o = jnp.dot(h, w3_ref[...], preferred_element_type=f32)
        acc = jnp.maximum(acc, o)
    out_ref[...] = acc + b3_ref[...]


def _mlp_call(g, kn, W1, W1k, b1, W2, b2, W3, b3):
    nq = g.shape[1]
    grid = (nq // QB,)
    return pl.pallas_call(
        _mlp_body,
        grid=grid,
        in_specs=[
            pl.BlockSpec((NB, QB, IN_SIZE), lambda i: (0, i, 0)),
            pl.BlockSpec((QB, 8), lambda i: (i, 0)),
            pl.BlockSpec((IN_SIZE, H1), lambda i: (0, 0)),
            pl.BlockSpec((8, H1), lambda i: (0, 0)),
            pl.BlockSpec((1, H1), lambda i: (0, 0)),
            pl.BlockSpec((H1, H2), lambda i: (0, 0)),
            pl.BlockSpec((1, H2), lambda i: (0, 0)),
            pl.BlockSpec((H2, C_OUT), lambda i: (0, 0)),
            pl.BlockSpec((1, C_OUT), lambda i: (0, 0)),
        ],
        out_specs=pl.BlockSpec((QB, C_OUT), lambda i: (i, 0)),
        out_shape=jax.ShapeDtypeStruct((nq, C_OUT), jnp.float32),
    )(g, kn, W1, W1k, b1, W2, b2, W3, b3)


def kernel(keys, points, feats, W1, b1, W2, b2, W3, b3):
    f32 = jnp.float32
    nb = B // SPLIT
    tpb = NW // nb
    qpw = nb * K // NW

    W1k = jnp.concatenate([W1[:DIM], jnp.zeros((8 - DIM, H1), f32)], axis=0)
    b1r, b2r, b3r = b1.reshape(1, H1), b2.reshape(1, H2), b3.reshape(1, C_OUT)

    outs = []
    for h in range(SPLIT):
        kh = lax.slice_in_dim(keys, h * nb, (h + 1) * nb, axis=0)
        ph = lax.slice_in_dim(points, h * nb, (h + 1) * nb, axis=0)
        fh = lax.slice_in_dim(feats, h * nb, (h + 1) * nb, axis=0)
        nq = nb * K
        pts_t = ph.transpose(0, 2, 1).reshape(nb, DIM * N)
        keys_g = (kh.reshape(nb, tpb, qpw, DIM)
                  .transpose(0, 1, 3, 2)
                  .reshape(NW, DIM * qpw))
        table = jnp.concatenate([ph, fh], axis=2).reshape(nb * N, IN_SIZE)
        g = _fused_call(pts_t, keys_g, table, nb).reshape(NB, nq, IN_SIZE)
        kflat = kh.reshape(nq, DIM)
        kn = jnp.concatenate([-kflat, jnp.zeros((nq, 8 - DIM), f32)], axis=1)
        outs.append(_mlp_call(g, kn, W1, W1k, b1r, W2, b2r, W3, b3r))

    out = jnp.concatenate(outs, axis=0) if SPLIT > 1 else outs[0]
    return out.reshape(B, K, C_OUT)
